# baseline (speedup 1.0000x reference)
import functools

import jax
import jax.numpy as jnp
from jax import lax
from jax.experimental import pallas as pl
from jax.experimental.pallas import tpu as pltpu
from jax.experimental.pallas import tpu_sc as plsc

_B = 131072
_K = 512
_D = 64
_COMMIT = 0.9

_BLK = 2048
_NBLK = _B // _BLK


def _argmin_body(x_ref, cb_ref, idx_ref, loss_ref):
    i = pl.program_id(0)
    x = x_ref[...]
    cb = cb_ref[...]
    x2 = jnp.sum(x * x, axis=1, keepdims=True)
    e2 = jnp.sum(cb * cb, axis=1)
    mm = lax.dot_general(x, cb, (((1,), (1,)), ((), ())),
                         preferred_element_type=jnp.float32)
    dist = (x2 + e2[None, :]) - 2.0 * mm
    minv = jnp.min(dist, axis=1, keepdims=True)
    ks = lax.broadcasted_iota(jnp.int32, (_BLK, _K), 1)
    idx = jnp.min(jnp.where(dist == minv, ks, _K), axis=1)
    idx_ref[...] = idx
    s = jnp.sum(minv)

    @pl.when(i == 0)
    def _():
        loss_ref[0, 0] = s

    @pl.when(i > 0)
    def _():
        loss_ref[0, 0] += s

    @pl.when(i == _NBLK - 1)
    def _():
        loss_ref[0, 0] = loss_ref[0, 0] * ((1.0 + _COMMIT) / (_B * _D))


_argmin_call = pl.pallas_call(
    _argmin_body,
    grid=(_NBLK,),
    in_specs=[
        pl.BlockSpec((_BLK, _D), lambda i: (i, 0)),
        pl.BlockSpec((_K, _D), lambda i: (0, 0)),
    ],
    out_specs=[
        pl.BlockSpec((_BLK,), lambda i: (i,)),
        pl.BlockSpec(memory_space=pltpu.SMEM),
    ],
    out_shape=[
        jax.ShapeDtypeStruct((_B,), jnp.int32),
        jax.ShapeDtypeStruct((1, 1), jnp.float32),
    ],
)

_NUM_SC_CORES = 2
_NUM_SC_SUBCORES = 16
_NW = _NUM_SC_CORES * _NUM_SC_SUBCORES
_BPW = _B // _NW
_CH = 128
_NCH = _BPW // _CH


def _sc_gather_body(cb_hbm, idx_hbm, out_hbm, idx_v, rows_v, sem):
    wid = lax.axis_index("s") * _NUM_SC_CORES + lax.axis_index("c")
    base = wid * _BPW
    pltpu.sync_copy(idx_hbm.at[pl.ds(base, _BPW)], idx_v)

    def body(j, carry):
        pltpu.async_copy(cb_hbm.at[idx_v.at[pl.ds(j * _CH, _CH)]],
                         rows_v, sem).wait()
        pltpu.sync_copy(rows_v, out_hbm.at[pl.ds(base + j * _CH, _CH)])
        return carry

    lax.fori_loop(0, _NCH, body, 0)


@functools.cache
def _sc_gather_call():
    return pl.kernel(
        _sc_gather_body,
        mesh=plsc.VectorSubcoreMesh(core_axis_name="c", subcore_axis_name="s"),
        out_type=jax.ShapeDtypeStruct((_B, _D), jnp.float32),
        scratch_types=[
            pltpu.VMEM((_BPW,), jnp.int32),
            pltpu.VMEM((_CH, _D), jnp.float32),
            pltpu.SemaphoreType.DMA,
        ],
        compiler_params=pltpu.CompilerParams(use_tc_tiling_on_sc=False),
    )


def kernel(inputs, codebook):
    idx, loss = _argmin_call(inputs, codebook)
    quantized = _sc_gather_call()(codebook, idx)
    return quantized, loss.reshape(()), idx


# speedup vs baseline: 1.2432x; 1.2432x over previous
import functools

import jax
import jax.numpy as jnp
from jax import lax
from jax.experimental import pallas as pl
from jax.experimental.pallas import tpu as pltpu
from jax.experimental.pallas import tpu_sc as plsc

_B = 131072
_K = 512
_D = 64
_COMMIT = 0.9

_BLK = 4096
_NBLK = _B // _BLK


def _argmin_body(x_ref, cb_ref, idx_ref, loss_ref):
    i = pl.program_id(0)
    x = x_ref[...]
    cb = cb_ref[...]
    ones_row = jnp.ones((1, _D), jnp.float32)
    x2 = lax.dot_general(ones_row, x * x, (((1,), (1,)), ((), ())),
                         preferred_element_type=jnp.float32)
    e2 = lax.dot_general(cb * cb, ones_row, (((1,), (1,)), ((), ())),
                         preferred_element_type=jnp.float32)
    mm = lax.dot_general(cb, x, (((1,), (1,)), ((), ())),
                         preferred_element_type=jnp.float32)
    dist = (x2 + e2) - 2.0 * mm
    minv = jnp.min(dist, axis=0, keepdims=True)
    ks = lax.broadcasted_iota(jnp.int32, (_K, _BLK), 0)
    idx = jnp.min(jnp.where(dist == minv, ks, _K), axis=0)
    idx_ref[...] = idx
    s = jnp.sum(minv)

    @pl.when(i == 0)
    def _():
        loss_ref[0, 0] = s

    @pl.when(i > 0)
    def _():
        loss_ref[0, 0] += s

    @pl.when(i == _NBLK - 1)
    def _():
        loss_ref[0, 0] = loss_ref[0, 0] * ((1.0 + _COMMIT) / (_B * _D))


_argmin_call = pl.pallas_call(
    _argmin_body,
    grid=(_NBLK,),
    in_specs=[
        pl.BlockSpec((_BLK, _D), lambda i: (i, 0)),
        pl.BlockSpec((_K, _D), lambda i: (0, 0)),
    ],
    out_specs=[
        pl.BlockSpec((_BLK,), lambda i: (i,)),
        pl.BlockSpec(memory_space=pltpu.SMEM),
    ],
    out_shape=[
        jax.ShapeDtypeStruct((_B,), jnp.int32),
        jax.ShapeDtypeStruct((1, 1), jnp.float32),
    ],
)

_NUM_SC_CORES = 2
_NUM_SC_SUBCORES = 16
_NW = _NUM_SC_CORES * _NUM_SC_SUBCORES
_BPW = _B // _NW
_CH = 128
_NCH = _BPW // _CH


def _sc_gather_body(cb_hbm, idx_hbm, out_hbm, idx_v, rows_v, sem):
    wid = lax.axis_index("s") * _NUM_SC_CORES + lax.axis_index("c")
    base = wid * _BPW
    pltpu.sync_copy(idx_hbm.at[pl.ds(base, _BPW)], idx_v)

    def body(j, carry):
        pltpu.async_copy(cb_hbm.at[idx_v.at[pl.ds(j * _CH, _CH)]],
                         rows_v, sem).wait()
        pltpu.sync_copy(rows_v, out_hbm.at[pl.ds(base + j * _CH, _CH)])
        return carry

    lax.fori_loop(0, _NCH, body, 0)


@functools.cache
def _sc_gather_call():
    return pl.kernel(
        _sc_gather_body,
        mesh=plsc.VectorSubcoreMesh(core_axis_name="c", subcore_axis_name="s"),
        out_type=jax.ShapeDtypeStruct((_B, _D), jnp.float32),
        scratch_types=[
            pltpu.VMEM((_BPW,), jnp.int32),
            pltpu.VMEM((_CH, _D), jnp.float32),
            pltpu.SemaphoreType.DMA,
        ],
        compiler_params=pltpu.CompilerParams(use_tc_tiling_on_sc=False),
    )


def kernel(inputs, codebook):
    idx, loss = _argmin_call(inputs, codebook)
    quantized = _sc_gather_call()(codebook, idx)
    return quantized, loss.reshape(()), idx


# speedup vs baseline: 1.2529x; 1.0078x over previous
import functools

import jax
import jax.numpy as jnp
from jax import lax
from jax.experimental import pallas as pl
from jax.experimental.pallas import tpu as pltpu
from jax.experimental.pallas import tpu_sc as plsc

_B = 131072
_K = 512
_D = 64
_COMMIT = 0.9

_BLK = 4096
_NBLK = _B // _BLK


def _argmin_body(x_ref, cb_ref, idx_ref, loss_ref):
    i = pl.program_id(0)
    x = x_ref[...]
    cb = cb_ref[...]
    ones_row = jnp.ones((1, _D), jnp.float32)
    x2 = lax.dot_general(ones_row, x * x, (((1,), (1,)), ((), ())),
                         preferred_element_type=jnp.float32)
    e2 = lax.dot_general(cb * cb, ones_row, (((1,), (1,)), ((), ())),
                         preferred_element_type=jnp.float32)
    mm = lax.dot_general(cb, x, (((1,), (1,)), ((), ())),
                         preferred_element_type=jnp.float32)
    dist = (x2 + e2) - 2.0 * mm
    minv = jnp.min(dist, axis=0, keepdims=True)
    ks = lax.broadcasted_iota(jnp.int32, (_K, _BLK), 0)
    idx = jnp.min(jnp.where(dist == minv, ks, _K), axis=0)
    idx_ref[...] = idx
    s = jnp.sum(minv)

    @pl.when(i == 0)
    def _():
        loss_ref[0, 0] = s

    @pl.when(i > 0)
    def _():
        loss_ref[0, 0] += s

    @pl.when(i == _NBLK - 1)
    def _():
        loss_ref[0, 0] = loss_ref[0, 0] * ((1.0 + _COMMIT) / (_B * _D))


_argmin_call = pl.pallas_call(
    _argmin_body,
    grid=(_NBLK,),
    in_specs=[
        pl.BlockSpec((_BLK, _D), lambda i: (i, 0)),
        pl.BlockSpec((_K, _D), lambda i: (0, 0)),
    ],
    out_specs=[
        pl.BlockSpec((_BLK,), lambda i: (i,)),
        pl.BlockSpec(memory_space=pltpu.SMEM),
    ],
    out_shape=[
        jax.ShapeDtypeStruct((_B,), jnp.int32),
        jax.ShapeDtypeStruct((1, 1), jnp.float32),
    ],
)

_NUM_SC_CORES = 2
_NUM_SC_SUBCORES = 16
_NW = _NUM_SC_CORES * _NUM_SC_SUBCORES
_BPW = _B // _NW
_CH = 128
_NCH = _BPW // _CH


def _sc_gather_body(cb_hbm, idx_hbm, out_hbm, idx_v, rows0, rows1, sem0, sem1):
    wid = lax.axis_index("s") * _NUM_SC_CORES + lax.axis_index("c")
    base = wid * _BPW
    pltpu.sync_copy(idx_hbm.at[pl.ds(base, _BPW)], idx_v)

    def fire(j, buf, sem):
        return pltpu.async_copy(cb_hbm.at[idx_v.at[pl.ds(j * _CH, _CH)]],
                                buf, sem)

    fire(0, rows0, sem0)

    def body(p, carry):
        j = 2 * p
        fire(j + 1, rows1, sem1)
        pltpu.make_async_copy(cb_hbm.at[idx_v.at[pl.ds(j * _CH, _CH)]],
                              rows0, sem0).wait()
        pltpu.sync_copy(rows0, out_hbm.at[pl.ds(base + j * _CH, _CH)])

        @pl.when(j + 2 < _NCH)
        def _():
            fire(j + 2, rows0, sem0)

        pltpu.make_async_copy(cb_hbm.at[idx_v.at[pl.ds((j + 1) * _CH, _CH)]],
                              rows1, sem1).wait()
        pltpu.sync_copy(rows1, out_hbm.at[pl.ds(base + (j + 1) * _CH, _CH)])
        return carry

    lax.fori_loop(0, _NCH // 2, body, 0)


@functools.cache
def _sc_gather_call():
    return pl.kernel(
        _sc_gather_body,
        mesh=plsc.VectorSubcoreMesh(core_axis_name="c", subcore_axis_name="s"),
        out_type=jax.ShapeDtypeStruct((_B, _D), jnp.float32),
        scratch_types=[
            pltpu.VMEM((_BPW,), jnp.int32),
            pltpu.VMEM((_CH, _D), jnp.float32),
            pltpu.VMEM((_CH, _D), jnp.float32),
            pltpu.SemaphoreType.DMA,
            pltpu.SemaphoreType.DMA,
        ],
        compiler_params=pltpu.CompilerParams(use_tc_tiling_on_sc=False),
    )


def kernel(inputs, codebook):
    idx, loss = _argmin_call(inputs, codebook)
    quantized = _sc_gather_call()(codebook, idx)
    return quantized, loss.reshape(()), idx
